# Initial kernel scaffold; baseline (speedup 1.0000x reference)
import jax
import jax.numpy as jnp
from jax.experimental import pallas as pl


def kernel(x, edge_index, edge_attr, W_in, b_in, Wm0, bm0, Wu0, bu0, Wm1, bm1, Wu1, bu1, Wm2, bm2, Wu2, bu2, W_node, b_node, Wnl1, bnl1, Wnl2, bnl2):
    raise NotImplementedError("write your pallas kernel here")


# speedup vs baseline: 3.8160x; 3.8160x over previous
import functools

import jax
import jax.numpy as jnp
from jax import lax
from jax.experimental import pallas as pl
from jax.experimental.pallas import tpu as pltpu
from jax.experimental.pallas import tpu_sc as plsc

F32 = jnp.float32

NN = 50000
EE = 800000
H = 64
HH = 32
HQ = 16

NTILE = 16
NP = 50048
RPT = NP // NTILE
BLK = NP // 16

KC = 1024
SUB = 8
EPT = 50176
NCH = EPT // KC
EP = EPT * NTILE


def _dot(a, b):
    return jnp.dot(a, b, preferred_element_type=F32)


def _fmt_body(s_ref, d_ref, e_ref, so_ref, dg_ref, do_ref, eo_ref):
    i = pl.program_id(0)
    rid = lax.broadcasted_iota(jnp.int32, (128, 128), 0) + i * 128
    ok = rid < (EE // 128)
    so_ref[...] = jnp.where(ok, s_ref[...] * 8, 0)
    dg_ref[...] = jnp.where(ok, d_ref[...] * 8 + 4, NN * 8 + 4)
    do_ref[...] = jnp.where(ok, d_ref[...], NN)
    rid2 = lax.broadcasted_iota(jnp.int32, (128, 256), 0) + i * 128
    ear = e_ref[...].astype(jnp.bfloat16).astype(F32)
    eo_ref[...] = jnp.where(rid2 < (EE // 128), ear, 0.0)


@jax.jit
def _t_fmt(src2d, dst2d, ea2d):
    idx = jax.ShapeDtypeStruct((EP // 128, 128), jnp.int32)
    return pl.pallas_call(
        _fmt_body,
        grid=(EP // 128 // 128,),
        in_specs=[
            pl.BlockSpec((128, 128), lambda i: (i, 0)),
            pl.BlockSpec((128, 128), lambda i: (i, 0)),
            pl.BlockSpec((128, 256), lambda i: (i, 0)),
        ],
        out_specs=[
            pl.BlockSpec((128, 128), lambda i: (i, 0)),
            pl.BlockSpec((128, 128), lambda i: (i, 0)),
            pl.BlockSpec((128, 128), lambda i: (i, 0)),
            pl.BlockSpec((128, 256), lambda i: (i, 0)),
        ],
        out_shape=[
            idx, idx, idx,
            jax.ShapeDtypeStruct((EP // 128, 256), F32),
        ],
    )(src2d, dst2d, ea2d)


def _wr_body(wm_ref, wr_ref):
    wr_ref[...] = wm_ref[...].astype(jnp.bfloat16).astype(F32)


@jax.jit
def _t_wround(wm2):
    return pl.pallas_call(
        _wr_body,
        out_shape=jax.ShapeDtypeStruct((2, H), F32),
    )(wm2)


def _in_body(x_ref, w_ref, b_ref, o_ref):
    o_ref[...] = _dot(x_ref[...], w_ref[...]) + b_ref[...]


@jax.jit
def _t_in(x, w_in, b_in):
    return pl.pallas_call(
        _in_body,
        grid=(16,),
        in_specs=[
            pl.BlockSpec((BLK, 2), lambda i: (i, 0)),
            pl.BlockSpec((2, H), lambda i: (0, 0)),
            pl.BlockSpec((1, H), lambda i: (0, 0)),
        ],
        out_specs=pl.BlockSpec((BLK, H), lambda i: (i, 0)),
        out_shape=jax.ShapeDtypeStruct((NN, H), F32),
    )(x, w_in, b_in)


def _prep_body(h_ref, wm_ref, ab_ref):
    h = h_ref[...]
    ab_ref[:, 0:H] = _dot(h, wm_ref[0:H, :])
    ab_ref[:, H:2 * H] = _dot(h, wm_ref[H:2 * H, :])


@jax.jit
def _t_prep(h, wm):
    return pl.pallas_call(
        _prep_body,
        grid=(16,),
        in_specs=[
            pl.BlockSpec((BLK, H), lambda i: (i, 0)),
            pl.BlockSpec((2 * H + 2, H), lambda i: (0, 0)),
        ],
        out_specs=pl.BlockSpec((BLK, 2 * H), lambda i: (i, 0)),
        out_shape=jax.ShapeDtypeStruct((NP, 2 * H), F32),
    )(h, wm)


def _upd_body(h_ref, g0_ref, g1_ref, g2_ref, g3_ref, wu_ref, bu_ref, o_ref):
    acc = _dot(h_ref[...], wu_ref[0:H, :])
    for q, g in enumerate((g0_ref, g1_ref, g2_ref, g3_ref)):
        acc = acc + _dot(g[...], wu_ref[H + q * HQ:H + (q + 1) * HQ, :])
    o_ref[...] = jnp.maximum(acc + bu_ref[...], 0.0)


@jax.jit
def _t_upd(h, g0, g1, g2, g3, wu, bu):
    return pl.pallas_call(
        _upd_body,
        grid=(16,),
        in_specs=[
            pl.BlockSpec((BLK, H), lambda i: (i, 0)),
            pl.BlockSpec((BLK, HQ), lambda i: (i, 0)),
            pl.BlockSpec((BLK, HQ), lambda i: (i, 0)),
            pl.BlockSpec((BLK, HQ), lambda i: (i, 0)),
            pl.BlockSpec((BLK, HQ), lambda i: (i, 0)),
            pl.BlockSpec((2 * H, H), lambda i: (0, 0)),
            pl.BlockSpec((1, H), lambda i: (0, 0)),
        ],
        out_specs=pl.BlockSpec((BLK, H), lambda i: (i, 0)),
        out_shape=jax.ShapeDtypeStruct((NN, H), F32),
    )(h, g0, g1, g2, g3, wu, bu)


def _head_body(h_ref, wn_ref, bn_ref, w1_ref, b1_ref, w2_ref, b2_ref,
               p_ref, nl_ref, acc_ref):
    i = pl.program_id(0)
    h = h_ref[...]
    logits = _dot(h, wn_ref[...]) + bn_ref[...]
    p_ref[...] = 1.0 / (1.0 + jnp.exp(-logits))
    rid = lax.broadcasted_iota(jnp.int32, (BLK, H), 0) + i * BLK
    hm = jnp.where(rid < NN, h, 0.0)
    part = jnp.sum(hm, axis=0, keepdims=True)

    @pl.when(i == 0)
    def _():
        acc_ref[...] = part

    @pl.when(i > 0)
    def _():
        acc_ref[...] = acc_ref[...] + part

    @pl.when(i == pl.num_programs(0) - 1)
    def _():
        hg = acc_ref[...] * (1.0 / NN)
        t = jnp.maximum(_dot(hg, w1_ref[...]) + b1_ref[...], 0.0)
        nl_ref[...] = _dot(t, w2_ref[...]) + b2_ref[...]


@jax.jit
def _t_head(h, wn, bn, w1, b1, w2, b2):
    return pl.pallas_call(
        _head_body,
        grid=(16,),
        in_specs=[
            pl.BlockSpec((BLK, H), lambda i: (i, 0)),
            pl.BlockSpec((H, 1), lambda i: (0, 0)),
            pl.BlockSpec((1, 1), lambda i: (0, 0)),
            pl.BlockSpec((H, HH), lambda i: (0, 0)),
            pl.BlockSpec((1, HH), lambda i: (0, 0)),
            pl.BlockSpec((HH, 4), lambda i: (0, 0)),
            pl.BlockSpec((1, 4), lambda i: (0, 0)),
        ],
        out_specs=[
            pl.BlockSpec((BLK, 1), lambda i: (i, 0)),
            pl.BlockSpec((1, 4), lambda i: (0, 0)),
        ],
        out_shape=[
            jax.ShapeDtypeStruct((NN, 1), F32),
            jax.ShapeDtypeStruct((1, 4), F32),
        ],
        scratch_shapes=[pltpu.VMEM((1, H), F32)],
    )(h, wn, bn, w1, b1, w2, b2)


def _sc_pass(abr, src2, dst2g, dst2, ea, wv, bv, out, src_v, dst_vg, dst_v,
             ea_v, ar, br, acc, sem_a, sem_b, q, s):
    we0 = wv[0, pl.ds(q * HQ, 16)]
    we1 = wv[1, pl.ds(q * HQ, 16)]
    bmv = bv[0, pl.ds(q * HQ, 16)]

    def _zb(j, c):
        ar[j, pl.ds(0, 16)] = jnp.zeros((16,), F32)
        return c

    lax.fori_loop(0, KC, _zb, 0)
    r0 = s * RPT
    for k in range(3):
        pltpu.sync_copy(ar.at[pl.ds(0, KC)], acc.at[pl.ds(r0 + k * KC, KC)])
    pltpu.sync_copy(ar.at[pl.ds(0, RPT - 3 * KC)],
                    acc.at[pl.ds(r0 + 3 * KC, RPT - 3 * KC)])
    plsc.subcore_barrier()

    ebase = s * EPT
    rbase = s * (EPT // 128)

    def _chunk(i, c):
        eb = ebase + i * KC
        rb = rbase + i * SUB
        pltpu.sync_copy(src2.at[pl.ds(rb, SUB)], src_v)
        pltpu.sync_copy(dst2g.at[pl.ds(rb, SUB)], dst_vg)
        pltpu.sync_copy(dst2.at[pl.ds(rb, SUB)], dst_v)
        pltpu.sync_copy(ea.at[pl.ds(2 * eb, 2 * KC)], ea_v)
        for r in range(SUB):
            for l in range(SUB):
                src_v[r, pl.ds(l * 16, 16)] = src_v[r, pl.ds(l * 16, 16)] + q
                dst_vg[r, pl.ds(l * 16, 16)] = dst_vg[r, pl.ds(l * 16, 16)] + q
        cps = []
        for j in range(SUB):
            cps.append(pltpu.async_copy(
                abr.at[src_v.at[j]], ar.at[pl.ds(j * 128, 128)], sem_a))
            cps.append(pltpu.async_copy(
                abr.at[dst_vg.at[j]], br.at[pl.ds(j * 128, 128)], sem_b))
        for d in cps:
            d.wait()

        def _edge(t, cc):
            ev = ea_v[pl.ds(t * 16, 16)]
            for u in range(8):
                j = t * 8 + u
                e0 = ev[2 * u]
                e1 = ev[2 * u + 1]
                m = ar[j, pl.ds(0, 16)] + br[j, pl.ds(0, 16)] \
                    + e0 * we0 + e1 * we1 + bmv
                ar[j, pl.ds(0, 16)] = jnp.maximum(m, 0.0)
            return cc

        lax.fori_loop(0, KC // 8, _edge, 0)
        for j in range(SUB):
            pltpu.sync_copy(ar.at[pl.ds(j * 128, 128)],
                            acc.at[dst_v.at[j]], add=True)
        return c

    lax.fori_loop(0, NCH, _chunk, 0)
    plsc.subcore_barrier()
    for k in range(3):
        pltpu.sync_copy(acc.at[pl.ds(r0 + k * KC, KC)],
                        out.at[pl.ds(r0 + k * KC, KC)])
    pltpu.sync_copy(acc.at[pl.ds(r0 + 3 * KC, RPT - 3 * KC)],
                    out.at[pl.ds(r0 + 3 * KC, RPT - 3 * KC)])


def _sc_body(ab, src2, dst2g, dst2, ea, wer, bm,
             g0, g1, g2, g3,
             src_v, dst_vg, dst_v, ea_v, ar, br, wv, bv, acc, sem_a, sem_b):
    c = lax.axis_index("c")
    s = lax.axis_index("s")
    pltpu.sync_copy(wer, wv)
    pltpu.sync_copy(bm, bv)
    abr = ab

    def _pass(q, g):
        _sc_pass(abr, src2, dst2g, dst2, ea, wv, bv, g, src_v, dst_vg,
                 dst_v, ea_v, ar, br, acc, sem_a, sem_b, q, s)

    @pl.when(c == 0)
    def _():
        _pass(0, g0)
        _pass(1, g1)

    @pl.when(c == 1)
    def _():
        _pass(2, g2)
        _pass(3, g3)


@jax.jit
def _sc_edge(ab8, src2, dst2g, dst2, ea, wer, bm):
    quarter = jax.ShapeDtypeStruct((NP, HQ), F32)
    mesh = plsc.VectorSubcoreMesh(core_axis_name="c", subcore_axis_name="s")
    kfn = pl.kernel(
        _sc_body,
        out_type=(quarter, quarter, quarter, quarter),
        mesh=mesh,
        scratch_types=[
            pltpu.VMEM((SUB, 128), jnp.int32),
            pltpu.VMEM((SUB, 128), jnp.int32),
            pltpu.VMEM((SUB, 128), jnp.int32),
            pltpu.VMEM((2 * KC,), F32),
            pltpu.VMEM((KC, HQ), F32),
            pltpu.VMEM((KC, HQ), F32),
            pltpu.VMEM((2, H), F32),
            pltpu.VMEM((1, H), F32),
            pltpu.VMEM_SHARED((NP, HQ), F32),
            pltpu.SemaphoreType.DMA,
            pltpu.SemaphoreType.DMA,
        ],
        compiler_params=pltpu.CompilerParams(use_tc_tiling_on_sc=False),
    )
    return kfn(ab8, src2, dst2g, dst2, ea, wer, bm)


def kernel(x, edge_index, edge_attr, W_in, b_in,
           Wm0, bm0, Wu0, bu0, Wm1, bm1, Wu1, bu1, Wm2, bm2, Wu2, bu2,
           W_node, b_node, Wnl1, bnl1, Wnl2, bnl2):
    src2d = edge_index[0].reshape(EE // 128, 128)
    dst2d = edge_index[1].reshape(EE // 128, 128)
    ea2d = edge_attr.reshape(EE // 128, 256)
    src8, dst8, dstp, ea2 = _t_fmt(src2d, dst2d, ea2d)
    eap = ea2.reshape(2 * EP)

    h = _t_in(x, W_in, b_in.reshape(1, H))
    for (wm, bm, wu, bu) in ((Wm0, bm0, Wu0, bu0), (Wm1, bm1, Wu1, bu1),
                             (Wm2, bm2, Wu2, bu2)):
        wer = _t_wround(wm[2 * H:])
        ab8 = _t_prep(h, wm).reshape(8 * NP, HQ)
        g0, g1, g2, g3 = _sc_edge(ab8, src8, dst8, dstp, eap, wer,
                                  bm.reshape(1, H))
        h = _t_upd(h, g0, g1, g2, g3, wu, bu.reshape(1, H))

    probs, nl = _t_head(h, W_node, b_node.reshape(1, 1),
                        Wnl1, bnl1.reshape(1, HH), Wnl2, bnl2.reshape(1, 4))
    return (probs.reshape(NN), nl, h)


# speedup vs baseline: 4.3790x; 1.1475x over previous
import functools

import jax
import jax.numpy as jnp
from jax import lax
from jax.experimental import pallas as pl
from jax.experimental.pallas import tpu as pltpu
from jax.experimental.pallas import tpu_sc as plsc

F32 = jnp.float32

NN = 50000
EE = 800000
H = 64
HH = 32
HQ = 16

NTILE = 16
NP = 50048
RPT = NP // NTILE
BLK = NP // 16

KC = 1024
SUB = 8
EPT = 50176
NCH = EPT // KC
EP = EPT * NTILE


def _dot(a, b):
    return jnp.dot(a, b, preferred_element_type=F32)


def _fmt_body(s_ref, d_ref, e_ref, so_ref, dg_ref, do_ref, eo_ref):
    i = pl.program_id(0)
    rid = lax.broadcasted_iota(jnp.int32, (128, 128), 0) + i * 128
    ok = rid < (EE // 128)
    so_ref[...] = jnp.where(ok, s_ref[...] * 8, 0)
    dg_ref[...] = jnp.where(ok, d_ref[...] * 8 + 4, NN * 8 + 4)
    do_ref[...] = jnp.where(ok, d_ref[...], NN)
    rid2 = lax.broadcasted_iota(jnp.int32, (128, 256), 0) + i * 128
    ear = e_ref[...].astype(jnp.bfloat16).astype(F32)
    eo_ref[...] = jnp.where(rid2 < (EE // 128), ear, 0.0)


@jax.jit
def _t_fmt(src2d, dst2d, ea2d):
    idx = jax.ShapeDtypeStruct((EP // 128, 128), jnp.int32)
    return pl.pallas_call(
        _fmt_body,
        grid=(EP // 128 // 128,),
        in_specs=[
            pl.BlockSpec((128, 128), lambda i: (i, 0)),
            pl.BlockSpec((128, 128), lambda i: (i, 0)),
            pl.BlockSpec((128, 256), lambda i: (i, 0)),
        ],
        out_specs=[
            pl.BlockSpec((128, 128), lambda i: (i, 0)),
            pl.BlockSpec((128, 128), lambda i: (i, 0)),
            pl.BlockSpec((128, 128), lambda i: (i, 0)),
            pl.BlockSpec((128, 256), lambda i: (i, 0)),
        ],
        out_shape=[
            idx, idx, idx,
            jax.ShapeDtypeStruct((EP // 128, 256), F32),
        ],
    )(src2d, dst2d, ea2d)


def _wr_body(wm_ref, wr_ref):
    wr_ref[...] = wm_ref[...].astype(jnp.bfloat16).astype(F32)


@jax.jit
def _t_wround(wm2):
    return pl.pallas_call(
        _wr_body,
        out_shape=jax.ShapeDtypeStruct((2, H), F32),
    )(wm2)


def _in_body(x_ref, w_ref, b_ref, o_ref):
    o_ref[...] = _dot(x_ref[...], w_ref[...]) + b_ref[...]


@jax.jit
def _t_in(x, w_in, b_in):
    return pl.pallas_call(
        _in_body,
        grid=(16,),
        in_specs=[
            pl.BlockSpec((BLK, 2), lambda i: (i, 0)),
            pl.BlockSpec((2, H), lambda i: (0, 0)),
            pl.BlockSpec((1, H), lambda i: (0, 0)),
        ],
        out_specs=pl.BlockSpec((BLK, H), lambda i: (i, 0)),
        out_shape=jax.ShapeDtypeStruct((NN, H), F32),
    )(x, w_in, b_in)


def _prep_body(h_ref, wm_ref, ab_ref):
    h = h_ref[...]
    ab_ref[:, 0:H] = _dot(h, wm_ref[0:H, :])
    ab_ref[:, H:2 * H] = _dot(h, wm_ref[H:2 * H, :])


@jax.jit
def _t_prep(h, wm):
    return pl.pallas_call(
        _prep_body,
        grid=(16,),
        in_specs=[
            pl.BlockSpec((BLK, H), lambda i: (i, 0)),
            pl.BlockSpec((2 * H + 2, H), lambda i: (0, 0)),
        ],
        out_specs=pl.BlockSpec((BLK, 2 * H), lambda i: (i, 0)),
        out_shape=jax.ShapeDtypeStruct((NP, 2 * H), F32),
    )(h, wm)


def _upd_body(h_ref, g0_ref, g1_ref, g2_ref, g3_ref, wu_ref, bu_ref, o_ref):
    acc = _dot(h_ref[...], wu_ref[0:H, :])
    for q, g in enumerate((g0_ref, g1_ref, g2_ref, g3_ref)):
        acc = acc + _dot(g[...], wu_ref[H + q * HQ:H + (q + 1) * HQ, :])
    o_ref[...] = jnp.maximum(acc + bu_ref[...], 0.0)


@jax.jit
def _t_upd(h, g0, g1, g2, g3, wu, bu):
    return pl.pallas_call(
        _upd_body,
        grid=(16,),
        in_specs=[
            pl.BlockSpec((BLK, H), lambda i: (i, 0)),
            pl.BlockSpec((BLK, HQ), lambda i: (i, 0)),
            pl.BlockSpec((BLK, HQ), lambda i: (i, 0)),
            pl.BlockSpec((BLK, HQ), lambda i: (i, 0)),
            pl.BlockSpec((BLK, HQ), lambda i: (i, 0)),
            pl.BlockSpec((2 * H, H), lambda i: (0, 0)),
            pl.BlockSpec((1, H), lambda i: (0, 0)),
        ],
        out_specs=pl.BlockSpec((BLK, H), lambda i: (i, 0)),
        out_shape=jax.ShapeDtypeStruct((NN, H), F32),
    )(h, g0, g1, g2, g3, wu, bu)


def _head_body(h_ref, wn_ref, bn_ref, w1_ref, b1_ref, w2_ref, b2_ref,
               p_ref, nl_ref, acc_ref):
    i = pl.program_id(0)
    h = h_ref[...]
    logits = _dot(h, wn_ref[...]) + bn_ref[...]
    p_ref[...] = 1.0 / (1.0 + jnp.exp(-logits))
    rid = lax.broadcasted_iota(jnp.int32, (BLK, H), 0) + i * BLK
    hm = jnp.where(rid < NN, h, 0.0)
    part = jnp.sum(hm, axis=0, keepdims=True)

    @pl.when(i == 0)
    def _():
        acc_ref[...] = part

    @pl.when(i > 0)
    def _():
        acc_ref[...] = acc_ref[...] + part

    @pl.when(i == pl.num_programs(0) - 1)
    def _():
        hg = acc_ref[...] * (1.0 / NN)
        t = jnp.maximum(_dot(hg, w1_ref[...]) + b1_ref[...], 0.0)
        nl_ref[...] = _dot(t, w2_ref[...]) + b2_ref[...]


@jax.jit
def _t_head(h, wn, bn, w1, b1, w2, b2):
    return pl.pallas_call(
        _head_body,
        grid=(16,),
        in_specs=[
            pl.BlockSpec((BLK, H), lambda i: (i, 0)),
            pl.BlockSpec((H, 1), lambda i: (0, 0)),
            pl.BlockSpec((1, 1), lambda i: (0, 0)),
            pl.BlockSpec((H, HH), lambda i: (0, 0)),
            pl.BlockSpec((1, HH), lambda i: (0, 0)),
            pl.BlockSpec((HH, 4), lambda i: (0, 0)),
            pl.BlockSpec((1, 4), lambda i: (0, 0)),
        ],
        out_specs=[
            pl.BlockSpec((BLK, 1), lambda i: (i, 0)),
            pl.BlockSpec((1, 4), lambda i: (0, 0)),
        ],
        out_shape=[
            jax.ShapeDtypeStruct((NN, 1), F32),
            jax.ShapeDtypeStruct((1, 4), F32),
        ],
        scratch_shapes=[pltpu.VMEM((1, H), F32)],
    )(h, wn, bn, w1, b1, w2, b2)


def _sc_pass(abr, src2, dst2g, dst2, ea, wv, bv, out, sv, dgv, dv,
             eav, arr, brr, acc, sem_a, sem_b, q, s):
    we0 = wv[0, pl.ds(q * HQ, 16)]
    we1 = wv[1, pl.ds(q * HQ, 16)]
    bmv = bv[0, pl.ds(q * HQ, 16)]

    def _zb(j, c):
        arr[0][j, pl.ds(0, 16)] = jnp.zeros((16,), F32)
        return c

    lax.fori_loop(0, KC, _zb, 0)
    r0 = s * RPT
    for k in range(3):
        pltpu.sync_copy(arr[0].at[pl.ds(0, KC)],
                        acc.at[pl.ds(r0 + k * KC, KC)])
    pltpu.sync_copy(arr[0].at[pl.ds(0, RPT - 3 * KC)],
                    acc.at[pl.ds(r0 + 3 * KC, RPT - 3 * KC)])
    plsc.subcore_barrier()

    ebase = s * EPT
    rbase = s * (EPT // 128)

    def _load_and_fire(c, p):
        eb = ebase + c * KC
        rb = rbase + c * SUB
        pltpu.sync_copy(src2.at[pl.ds(rb, SUB)], sv[p])
        pltpu.sync_copy(dst2g.at[pl.ds(rb, SUB)], dgv[p])
        pltpu.sync_copy(dst2.at[pl.ds(rb, SUB)], dv[p])
        pltpu.sync_copy(ea.at[pl.ds(2 * eb, 2 * KC)], eav[p])
        for r in range(SUB):
            for l in range(SUB):
                sv[p][r, pl.ds(l * 16, 16)] = sv[p][r, pl.ds(l * 16, 16)] + q
                dgv[p][r, pl.ds(l * 16, 16)] = \
                    dgv[p][r, pl.ds(l * 16, 16)] + q
        for j in range(SUB):
            pltpu.async_copy(abr.at[sv[p].at[j]],
                             arr[p].at[pl.ds(j * 128, 128)], sem_a)
            pltpu.async_copy(abr.at[dgv[p].at[j]],
                             brr[p].at[pl.ds(j * 128, 128)], sem_b)

    def _drain(p):
        for j in range(SUB):
            pltpu.make_async_copy(abr.at[sv[p].at[j]],
                                  arr[p].at[pl.ds(j * 128, 128)],
                                  sem_a).wait()
            pltpu.make_async_copy(abr.at[dgv[p].at[j]],
                                  brr[p].at[pl.ds(j * 128, 128)],
                                  sem_b).wait()

    def _compute_scatter(p):
        ar = arr[p]
        br = brr[p]
        ea_v = eav[p]

        def _edge(t, cc):
            ev = ea_v[pl.ds(t * 16, 16)]
            for u in range(8):
                j = t * 8 + u
                e0 = ev[2 * u]
                e1 = ev[2 * u + 1]
                m = ar[j, pl.ds(0, 16)] + br[j, pl.ds(0, 16)] \
                    + e0 * we0 + e1 * we1 + bmv
                ar[j, pl.ds(0, 16)] = jnp.maximum(m, 0.0)
            return cc

        lax.fori_loop(0, KC // 8, _edge, 0)
        for j in range(SUB):
            pltpu.sync_copy(ar.at[pl.ds(j * 128, 128)],
                            acc.at[dv[p].at[j]], add=True)

    def _lf_chunk(c, p):
        @pl.when(c < NCH)
        def _():
            _load_and_fire(c, p)

    _load_and_fire(0, 0)

    def _body(io, c):
        for parity in (1, 0):
            cur = 2 * io + (1 - parity)
            _drain(1 - parity)
            _lf_chunk(cur + 1, parity)
            _compute_scatter(1 - parity)
        return c

    lax.fori_loop(0, NCH // 2, _body, 0)
    _drain(0)
    _compute_scatter(0)

    plsc.subcore_barrier()
    for k in range(3):
        pltpu.sync_copy(acc.at[pl.ds(r0 + k * KC, KC)],
                        out.at[pl.ds(r0 + k * KC, KC)])
    pltpu.sync_copy(acc.at[pl.ds(r0 + 3 * KC, RPT - 3 * KC)],
                    out.at[pl.ds(r0 + 3 * KC, RPT - 3 * KC)])


def _sc_body(ab, src2, dst2g, dst2, ea, wer, bm,
             g0, g1, g2, g3,
             sv0, sv1, dgv0, dgv1, dv0, dv1, eav0, eav1,
             ar0, ar1, br0, br1, wv, bv, acc, sem_a, sem_b):
    c = lax.axis_index("c")
    s = lax.axis_index("s")
    pltpu.sync_copy(wer, wv)
    pltpu.sync_copy(bm, bv)
    abr = ab

    def _pass(q, g):
        _sc_pass(abr, src2, dst2g, dst2, ea, wv, bv, g, (sv0, sv1),
                 (dgv0, dgv1), (dv0, dv1), (eav0, eav1), (ar0, ar1),
                 (br0, br1), acc, sem_a, sem_b, q, s)

    @pl.when(c == 0)
    def _():
        _pass(0, g0)
        _pass(1, g1)

    @pl.when(c == 1)
    def _():
        _pass(2, g2)
        _pass(3, g3)


@jax.jit
def _sc_edge(ab8, src2, dst2g, dst2, ea, wer, bm):
    quarter = jax.ShapeDtypeStruct((NP, HQ), F32)
    mesh = plsc.VectorSubcoreMesh(core_axis_name="c", subcore_axis_name="s")
    kfn = pl.kernel(
        _sc_body,
        out_type=(quarter, quarter, quarter, quarter),
        mesh=mesh,
        scratch_types=[
            pltpu.VMEM((SUB, 128), jnp.int32),
            pltpu.VMEM((SUB, 128), jnp.int32),
            pltpu.VMEM((SUB, 128), jnp.int32),
            pltpu.VMEM((SUB, 128), jnp.int32),
            pltpu.VMEM((SUB, 128), jnp.int32),
            pltpu.VMEM((SUB, 128), jnp.int32),
            pltpu.VMEM((2 * KC,), F32),
            pltpu.VMEM((2 * KC,), F32),
            pltpu.VMEM((KC, HQ), F32),
            pltpu.VMEM((KC, HQ), F32),
            pltpu.VMEM((KC, HQ), F32),
            pltpu.VMEM((KC, HQ), F32),
            pltpu.VMEM((2, H), F32),
            pltpu.VMEM((1, H), F32),
            pltpu.VMEM_SHARED((NP, HQ), F32),
            pltpu.SemaphoreType.DMA,
            pltpu.SemaphoreType.DMA,
        ],
        compiler_params=pltpu.CompilerParams(use_tc_tiling_on_sc=False),
    )
    return kfn(ab8, src2, dst2g, dst2, ea, wer, bm)


def kernel(x, edge_index, edge_attr, W_in, b_in,
           Wm0, bm0, Wu0, bu0, Wm1, bm1, Wu1, bu1, Wm2, bm2, Wu2, bu2,
           W_node, b_node, Wnl1, bnl1, Wnl2, bnl2):
    src2d = edge_index[0].reshape(EE // 128, 128)
    dst2d = edge_index[1].reshape(EE // 128, 128)
    ea2d = edge_attr.reshape(EE // 128, 256)
    src8, dst8, dstp, ea2 = _t_fmt(src2d, dst2d, ea2d)
    eap = ea2.reshape(2 * EP)

    h = _t_in(x, W_in, b_in.reshape(1, H))
    for (wm, bm, wu, bu) in ((Wm0, bm0, Wu0, bu0), (Wm1, bm1, Wu1, bu1),
                             (Wm2, bm2, Wu2, bu2)):
        wer = _t_wround(wm[2 * H:])
        ab8 = _t_prep(h, wm).reshape(8 * NP, HQ)
        g0, g1, g2, g3 = _sc_edge(ab8, src8, dst8, dstp, eap, wer,
                                  bm.reshape(1, H))
        h = _t_upd(h, g0, g1, g2, g3, wu, bu.reshape(1, H))

    probs, nl = _t_head(h, W_node, b_node.reshape(1, 1),
                        Wnl1, bnl1.reshape(1, HH), Wnl2, bnl2.reshape(1, 4))
    return (probs.reshape(NN), nl, h)


# speedup vs baseline: 4.5428x; 1.0374x over previous
import functools

import jax
import jax.numpy as jnp
from jax import lax
from jax.experimental import pallas as pl
from jax.experimental.pallas import tpu as pltpu
from jax.experimental.pallas import tpu_sc as plsc

F32 = jnp.float32

NN = 50000
EE = 800000
H = 64
HH = 32
HQ = 16

NTILE = 16
NP = 50048
RPT = NP // NTILE
BLK = NP // 16

KC = 1024
SUB = 8
EPT = 50176
NCH = EPT // KC
EP = EPT * NTILE


def _dot(a, b):
    return jnp.dot(a, b, preferred_element_type=F32)


def _fmt_body(s_ref, d_ref, e_ref, so_ref, dg_ref, do_ref, eo_ref):
    i = pl.program_id(0)
    rid = lax.broadcasted_iota(jnp.int32, (128, 128), 0) + i * 128
    ok = rid < (EE // 128)
    so_ref[...] = jnp.where(ok, s_ref[...] * 8, 0)
    dg_ref[...] = jnp.where(ok, d_ref[...] * 8 + 4, NN * 8 + 4)
    do_ref[...] = jnp.where(ok, d_ref[...], NN)
    rid2 = lax.broadcasted_iota(jnp.int32, (128, 256), 0) + i * 128
    ear = e_ref[...].astype(jnp.bfloat16).astype(F32)
    eo_ref[...] = jnp.where(rid2 < (EE // 128), ear, 0.0)


@jax.jit
def _t_fmt(src2d, dst2d, ea2d):
    idx = jax.ShapeDtypeStruct((EP // 128, 128), jnp.int32)
    return pl.pallas_call(
        _fmt_body,
        grid=(EP // 128 // 128,),
        in_specs=[
            pl.BlockSpec((128, 128), lambda i: (i, 0)),
            pl.BlockSpec((128, 128), lambda i: (i, 0)),
            pl.BlockSpec((128, 256), lambda i: (i, 0)),
        ],
        out_specs=[
            pl.BlockSpec((128, 128), lambda i: (i, 0)),
            pl.BlockSpec((128, 128), lambda i: (i, 0)),
            pl.BlockSpec((128, 128), lambda i: (i, 0)),
            pl.BlockSpec((128, 256), lambda i: (i, 0)),
        ],
        out_shape=[
            idx, idx, idx,
            jax.ShapeDtypeStruct((EP // 128, 256), F32),
        ],
    )(src2d, dst2d, ea2d)


def _wr_body(wm_ref, wr_ref):
    wr_ref[...] = wm_ref[...].astype(jnp.bfloat16).astype(F32)


@jax.jit
def _t_wround(wm2):
    return pl.pallas_call(
        _wr_body,
        out_shape=jax.ShapeDtypeStruct((2, H), F32),
    )(wm2)


def _in_body(x_ref, w_ref, b_ref, o_ref):
    o_ref[...] = _dot(x_ref[...], w_ref[...]) + b_ref[...]


@jax.jit
def _t_in(x, w_in, b_in):
    return pl.pallas_call(
        _in_body,
        grid=(16,),
        in_specs=[
            pl.BlockSpec((BLK, 2), lambda i: (i, 0)),
            pl.BlockSpec((2, H), lambda i: (0, 0)),
            pl.BlockSpec((1, H), lambda i: (0, 0)),
        ],
        out_specs=pl.BlockSpec((BLK, H), lambda i: (i, 0)),
        out_shape=jax.ShapeDtypeStruct((NN, H), F32),
    )(x, w_in, b_in)


def _prep_body(h_ref, wm_ref, ab_ref):
    h = h_ref[...]
    ab_ref[:, 0:H] = _dot(h, wm_ref[0:H, :])
    ab_ref[:, H:2 * H] = _dot(h, wm_ref[H:2 * H, :])


@jax.jit
def _t_prep(h, wm):
    return pl.pallas_call(
        _prep_body,
        grid=(16,),
        in_specs=[
            pl.BlockSpec((BLK, H), lambda i: (i, 0)),
            pl.BlockSpec((2 * H + 2, H), lambda i: (0, 0)),
        ],
        out_specs=pl.BlockSpec((BLK, 2 * H), lambda i: (i, 0)),
        out_shape=jax.ShapeDtypeStruct((NP, 2 * H), F32),
    )(h, wm)


def _upd_body(h_ref, g0_ref, g1_ref, g2_ref, g3_ref, wu_ref, bu_ref, o_ref):
    acc = _dot(h_ref[...], wu_ref[0:H, :])
    for q, g in enumerate((g0_ref, g1_ref, g2_ref, g3_ref)):
        acc = acc + _dot(g[...], wu_ref[H + q * HQ:H + (q + 1) * HQ, :])
    o_ref[...] = jnp.maximum(acc + bu_ref[...], 0.0)


@jax.jit
def _t_upd(h, g0, g1, g2, g3, wu, bu):
    return pl.pallas_call(
        _upd_body,
        grid=(16,),
        in_specs=[
            pl.BlockSpec((BLK, H), lambda i: (i, 0)),
            pl.BlockSpec((BLK, HQ), lambda i: (i, 0)),
            pl.BlockSpec((BLK, HQ), lambda i: (i, 0)),
            pl.BlockSpec((BLK, HQ), lambda i: (i, 0)),
            pl.BlockSpec((BLK, HQ), lambda i: (i, 0)),
            pl.BlockSpec((2 * H, H), lambda i: (0, 0)),
            pl.BlockSpec((1, H), lambda i: (0, 0)),
        ],
        out_specs=pl.BlockSpec((BLK, H), lambda i: (i, 0)),
        out_shape=jax.ShapeDtypeStruct((NN, H), F32),
    )(h, g0, g1, g2, g3, wu, bu)


def _head_body(h_ref, wn_ref, bn_ref, w1_ref, b1_ref, w2_ref, b2_ref,
               p_ref, nl_ref, acc_ref):
    i = pl.program_id(0)
    h = h_ref[...]
    logits = _dot(h, wn_ref[...]) + bn_ref[...]
    p_ref[...] = 1.0 / (1.0 + jnp.exp(-logits))
    rid = lax.broadcasted_iota(jnp.int32, (BLK, H), 0) + i * BLK
    hm = jnp.where(rid < NN, h, 0.0)
    part = jnp.sum(hm, axis=0, keepdims=True)

    @pl.when(i == 0)
    def _():
        acc_ref[...] = part

    @pl.when(i > 0)
    def _():
        acc_ref[...] = acc_ref[...] + part

    @pl.when(i == pl.num_programs(0) - 1)
    def _():
        hg = acc_ref[...] * (1.0 / NN)
        t = jnp.maximum(_dot(hg, w1_ref[...]) + b1_ref[...], 0.0)
        nl_ref[...] = _dot(t, w2_ref[...]) + b2_ref[...]


@jax.jit
def _t_head(h, wn, bn, w1, b1, w2, b2):
    return pl.pallas_call(
        _head_body,
        grid=(16,),
        in_specs=[
            pl.BlockSpec((BLK, H), lambda i: (i, 0)),
            pl.BlockSpec((H, 1), lambda i: (0, 0)),
            pl.BlockSpec((1, 1), lambda i: (0, 0)),
            pl.BlockSpec((H, HH), lambda i: (0, 0)),
            pl.BlockSpec((1, HH), lambda i: (0, 0)),
            pl.BlockSpec((HH, 4), lambda i: (0, 0)),
            pl.BlockSpec((1, 4), lambda i: (0, 0)),
        ],
        out_specs=[
            pl.BlockSpec((BLK, 1), lambda i: (i, 0)),
            pl.BlockSpec((1, 4), lambda i: (0, 0)),
        ],
        out_shape=[
            jax.ShapeDtypeStruct((NN, 1), F32),
            jax.ShapeDtypeStruct((1, 4), F32),
        ],
        scratch_shapes=[pltpu.VMEM((1, H), F32)],
    )(h, wn, bn, w1, b1, w2, b2)


def _sc_pass(abr, src2, dst2g, dst2, ea, wv, bv, out, sv, dgv, dv,
             eav, arr, brr, acc, sem_a, sem_b, sem_s, q, s):
    we0 = wv[0, pl.ds(q * HQ, 16)]
    we1 = wv[1, pl.ds(q * HQ, 16)]
    bmv = bv[0, pl.ds(q * HQ, 16)]

    def _zb(j, c):
        arr[0][j, pl.ds(0, 16)] = jnp.zeros((16,), F32)
        return c

    lax.fori_loop(0, KC, _zb, 0)
    r0 = s * RPT
    for k in range(3):
        pltpu.sync_copy(arr[0].at[pl.ds(0, KC)],
                        acc.at[pl.ds(r0 + k * KC, KC)])
    pltpu.sync_copy(arr[0].at[pl.ds(0, RPT - 3 * KC)],
                    acc.at[pl.ds(r0 + 3 * KC, RPT - 3 * KC)])
    plsc.subcore_barrier()

    ebase = s * EPT
    rbase = s * (EPT // 128)

    def _load_and_fire(c, p):
        @pl.when(c >= 2)
        def _():
            for j in range(SUB):
                pltpu.make_async_copy(arr[p].at[pl.ds(j * 128, 128)],
                                      acc.at[dv[p].at[j]], sem_s[p]).wait()
        eb = ebase + c * KC
        rb = rbase + c * SUB
        pltpu.sync_copy(src2.at[pl.ds(rb, SUB)], sv[p])
        pltpu.sync_copy(dst2g.at[pl.ds(rb, SUB)], dgv[p])
        pltpu.sync_copy(dst2.at[pl.ds(rb, SUB)], dv[p])
        pltpu.sync_copy(ea.at[pl.ds(2 * eb, 2 * KC)], eav[p])
        for r in range(SUB):
            for l in range(SUB):
                sv[p][r, pl.ds(l * 16, 16)] = sv[p][r, pl.ds(l * 16, 16)] + q
                dgv[p][r, pl.ds(l * 16, 16)] = \
                    dgv[p][r, pl.ds(l * 16, 16)] + q
        for j in range(SUB):
            pltpu.async_copy(abr.at[sv[p].at[j]],
                             arr[p].at[pl.ds(j * 128, 128)], sem_a)
            pltpu.async_copy(abr.at[dgv[p].at[j]],
                             brr[p].at[pl.ds(j * 128, 128)], sem_b)

    def _drain(p):
        for j in range(SUB):
            pltpu.make_async_copy(abr.at[sv[p].at[j]],
                                  arr[p].at[pl.ds(j * 128, 128)],
                                  sem_a).wait()
            pltpu.make_async_copy(abr.at[dgv[p].at[j]],
                                  brr[p].at[pl.ds(j * 128, 128)],
                                  sem_b).wait()

    def _compute_scatter(p):
        ar = arr[p]
        br = brr[p]
        ea_v = eav[p]

        def _edge(t, cc):
            ev = ea_v[pl.ds(t * 16, 16)]
            for u in range(8):
                j = t * 8 + u
                e0 = ev[2 * u]
                e1 = ev[2 * u + 1]
                m = ar[j, pl.ds(0, 16)] + br[j, pl.ds(0, 16)] \
                    + e0 * we0 + e1 * we1 + bmv
                ar[j, pl.ds(0, 16)] = jnp.maximum(m, 0.0)
            return cc

        lax.fori_loop(0, KC // 8, _edge, 0)
        for j in range(SUB):
            pltpu.async_copy(ar.at[pl.ds(j * 128, 128)],
                             acc.at[dv[p].at[j]], sem_s[p], add=True)

    def _lf_chunk(c, p):
        @pl.when(c < NCH)
        def _():
            _load_and_fire(c, p)

    _load_and_fire(0, 0)

    def _body(io, c):
        for parity in (1, 0):
            cur = 2 * io + (1 - parity)
            _drain(1 - parity)
            _lf_chunk(cur + 1, parity)
            _compute_scatter(1 - parity)
        return c

    lax.fori_loop(0, NCH // 2, _body, 0)
    _drain(0)
    _compute_scatter(0)
    for p in (0, 1):
        for j in range(SUB):
            pltpu.make_async_copy(arr[p].at[pl.ds(j * 128, 128)],
                                  acc.at[dv[p].at[j]], sem_s[p]).wait()

    plsc.subcore_barrier()
    for k in range(3):
        pltpu.sync_copy(acc.at[pl.ds(r0 + k * KC, KC)],
                        out.at[pl.ds(r0 + k * KC, KC)])
    pltpu.sync_copy(acc.at[pl.ds(r0 + 3 * KC, RPT - 3 * KC)],
                    out.at[pl.ds(r0 + 3 * KC, RPT - 3 * KC)])


def _sc_body(ab, src2, dst2g, dst2, ea, wer, bm,
             g0, g1, g2, g3,
             sv0, sv1, dgv0, dgv1, dv0, dv1, eav0, eav1,
             ar0, ar1, br0, br1, wv, bv, acc, sem_a, sem_b, sem_s0, sem_s1):
    c = lax.axis_index("c")
    s = lax.axis_index("s")
    pltpu.sync_copy(wer, wv)
    pltpu.sync_copy(bm, bv)
    abr = ab

    def _pass(q, g):
        _sc_pass(abr, src2, dst2g, dst2, ea, wv, bv, g, (sv0, sv1),
                 (dgv0, dgv1), (dv0, dv1), (eav0, eav1), (ar0, ar1),
                 (br0, br1), acc, sem_a, sem_b, (sem_s0, sem_s1), q, s)

    @pl.when(c == 0)
    def _():
        _pass(0, g0)
        _pass(1, g1)

    @pl.when(c == 1)
    def _():
        _pass(2, g2)
        _pass(3, g3)


@jax.jit
def _sc_edge(ab8, src2, dst2g, dst2, ea, wer, bm):
    quarter = jax.ShapeDtypeStruct((NP, HQ), F32)
    mesh = plsc.VectorSubcoreMesh(core_axis_name="c", subcore_axis_name="s")
    kfn = pl.kernel(
        _sc_body,
        out_type=(quarter, quarter, quarter, quarter),
        mesh=mesh,
        scratch_types=[
            pltpu.VMEM((SUB, 128), jnp.int32),
            pltpu.VMEM((SUB, 128), jnp.int32),
            pltpu.VMEM((SUB, 128), jnp.int32),
            pltpu.VMEM((SUB, 128), jnp.int32),
            pltpu.VMEM((SUB, 128), jnp.int32),
            pltpu.VMEM((SUB, 128), jnp.int32),
            pltpu.VMEM((2 * KC,), F32),
            pltpu.VMEM((2 * KC,), F32),
            pltpu.VMEM((KC, HQ), F32),
            pltpu.VMEM((KC, HQ), F32),
            pltpu.VMEM((KC, HQ), F32),
            pltpu.VMEM((KC, HQ), F32),
            pltpu.VMEM((2, H), F32),
            pltpu.VMEM((1, H), F32),
            pltpu.VMEM_SHARED((NP, HQ), F32),
            pltpu.SemaphoreType.DMA,
            pltpu.SemaphoreType.DMA,
            pltpu.SemaphoreType.DMA,
            pltpu.SemaphoreType.DMA,
        ],
        compiler_params=pltpu.CompilerParams(use_tc_tiling_on_sc=False),
    )
    return kfn(ab8, src2, dst2g, dst2, ea, wer, bm)


def kernel(x, edge_index, edge_attr, W_in, b_in,
           Wm0, bm0, Wu0, bu0, Wm1, bm1, Wu1, bu1, Wm2, bm2, Wu2, bu2,
           W_node, b_node, Wnl1, bnl1, Wnl2, bnl2):
    src2d = edge_index[0].reshape(EE // 128, 128)
    dst2d = edge_index[1].reshape(EE // 128, 128)
    ea2d = edge_attr.reshape(EE // 128, 256)
    src8, dst8, dstp, ea2 = _t_fmt(src2d, dst2d, ea2d)
    eap = ea2.reshape(2 * EP)

    h = _t_in(x, W_in, b_in.reshape(1, H))
    for (wm, bm, wu, bu) in ((Wm0, bm0, Wu0, bu0), (Wm1, bm1, Wu1, bu1),
                             (Wm2, bm2, Wu2, bu2)):
        wer = _t_wround(wm[2 * H:])
        ab8 = _t_prep(h, wm).reshape(8 * NP, HQ)
        g0, g1, g2, g3 = _sc_edge(ab8, src8, dst8, dstp, eap, wer,
                                  bm.reshape(1, H))
        h = _t_upd(h, g0, g1, g2, g3, wu, bu.reshape(1, H))

    probs, nl = _t_head(h, W_node, b_node.reshape(1, 1),
                        Wnl1, bnl1.reshape(1, HH), Wnl2, bnl2.reshape(1, 4))
    return (probs.reshape(NN), nl, h)
